# baseline (speedup 1.0000x reference)
import functools

import jax
import jax.numpy as jnp
from jax import lax
from jax.experimental import pallas as pl
from jax.experimental.pallas import tpu as pltpu
from jax.experimental.pallas import tpu_sc as plsc

B = 16384
DIM = 128
GAMMA = 12.0

NC = 2
NS = 16
L = 16
NW = NC * NS
BPW = B // NW
CH = 64
NCH = BPW // CH
UNR = 4

_mesh = plsc.VectorSubcoreMesh(core_axis_name="c", subcore_axis_name="s")


@functools.partial(
    pl.kernel,
    mesh=_mesh,
    out_type=jax.ShapeDtypeStruct((B,), jnp.float32),
    scratch_types=[
        pltpu.VMEM((BPW, 3), jnp.int32),
        pltpu.VMEM((BPW,), jnp.int32),
        pltpu.VMEM((BPW,), jnp.int32),
        pltpu.VMEM((BPW,), jnp.int32),
        pltpu.VMEM((CH, DIM), jnp.float32),
        pltpu.VMEM((CH, DIM), jnp.float32),
        pltpu.VMEM((CH, DIM), jnp.float32),
        pltpu.VMEM((CH, DIM), jnp.float32),
        pltpu.VMEM((CH, DIM), jnp.float32),
        pltpu.VMEM((CH, DIM), jnp.float32),
        pltpu.VMEM((BPW,), jnp.float32),
        pltpu.SemaphoreType.DMA,
        pltpu.SemaphoreType.DMA,
    ],
    compiler_params=pltpu.CompilerParams(needs_layout_passes=False),
)
def _sc_score(sample_hbm, ent_hbm, rel_hbm, out_hbm,
              smp_v, hi_v, ri_v, ti_v, hb0, hb1, rb0, rb1, tb0, tb1, ob,
              sem0, sem1):
    cid = lax.axis_index("c")
    sid = lax.axis_index("s")
    wid = sid * NC + cid
    base = wid * BPW

    pltpu.async_copy(sample_hbm.at[pl.ds(base, BPW)], smp_v, sem0).wait()

    lanes = lax.iota(jnp.int32, L)

    def split_body(i, _):
        rows = lanes + i * L
        for col, dst in ((0, hi_v), (1, ri_v), (2, ti_v)):
            cols = jnp.full((L,), col, jnp.int32)
            dst[pl.ds(i * L, L)] = plsc.load_gather(smp_v, [rows, cols])
        return 0

    lax.fori_loop(0, BPW // L, split_body, 0)

    bufs = [(hb0, rb0, tb0, sem0), (hb1, rb1, tb1, sem1)]

    def start(c):
        hb, rb, tb, sem = bufs[c % 2]
        sl = pl.ds(c * CH, CH)
        return (
            pltpu.async_copy(ent_hbm.at[hi_v.at[sl]], hb, sem),
            pltpu.async_copy(rel_hbm.at[ri_v.at[sl]], rb, sem),
            pltpu.async_copy(ent_hbm.at[ti_v.at[sl]], tb, sem),
        )

    pending = start(0)
    for c in range(NCH):
        for hdl in pending:
            hdl.wait()
        if c + 1 < NCH:
            pending = start(c + 1)
        hb, rb, tb, _ = bufs[c % 2]

        def group_body(g, _, hb=hb, rb=rb, tb=tb, c=c):
            rows = lanes + g * L

            def dbody(i, accs):
                out = []
                for k in range(UNR):
                    cols = (lanes + (i * UNR + k)) & (DIM - 1)
                    h = plsc.load_gather(hb, [rows, cols])
                    r = plsc.load_gather(rb, [rows, cols])
                    t = plsc.load_gather(tb, [rows, cols])
                    out.append(accs[k] + jnp.abs(h + r - t))
                return tuple(out)

            zero = jnp.zeros((L,), jnp.float32)
            accs = lax.fori_loop(0, DIM // UNR, dbody, (zero,) * UNR)
            acc = (accs[0] + accs[1]) + (accs[2] + accs[3])
            ob[pl.ds(c * CH + g * L, L)] = GAMMA - acc
            return 0

        lax.fori_loop(0, CH // L, group_body, 0)

    pltpu.sync_copy(ob, out_hbm.at[pl.ds(base, BPW)])


def kernel(sample, entity_embedding, relation_embedding):
    score = _sc_score(sample, entity_embedding, relation_embedding)
    return score[:, None]


# speedup vs baseline: 1.1077x; 1.1077x over previous
import functools

import jax
import jax.numpy as jnp
from jax import lax
from jax.experimental import pallas as pl
from jax.experimental.pallas import tpu as pltpu
from jax.experimental.pallas import tpu_sc as plsc

B = 16384
DIM = 128
GAMMA = 12.0

NC = 2
NS = 16
L = 16
NW = NC * NS
BPW = B // NW
CH = 128
NCH = BPW // CH
UNR = 4

_mesh = plsc.VectorSubcoreMesh(core_axis_name="c", subcore_axis_name="s")


@functools.partial(
    pl.kernel,
    mesh=_mesh,
    out_type=jax.ShapeDtypeStruct((B,), jnp.float32),
    scratch_types=[
        pltpu.VMEM((BPW,), jnp.int32),
        pltpu.VMEM((BPW,), jnp.int32),
        pltpu.VMEM((BPW,), jnp.int32),
        pltpu.VMEM((CH, DIM), jnp.float32),
        pltpu.VMEM((CH, DIM), jnp.float32),
        pltpu.VMEM((CH, DIM), jnp.float32),
        pltpu.VMEM((CH, DIM), jnp.float32),
        pltpu.VMEM((CH, DIM), jnp.float32),
        pltpu.VMEM((CH, DIM), jnp.float32),
        pltpu.VMEM((BPW,), jnp.float32),
        pltpu.SemaphoreType.DMA,
        pltpu.SemaphoreType.DMA,
    ],
    compiler_params=pltpu.CompilerParams(needs_layout_passes=False),
)
def _sc_score(hi_hbm, ri_hbm, ti_hbm, ent_hbm, rel_hbm, out_hbm,
              hi_v, ri_v, ti_v, hb0, hb1, rb0, rb1, tb0, tb1, ob,
              sem0, sem1):
    cid = lax.axis_index("c")
    sid = lax.axis_index("s")
    wid = sid * NC + cid
    base = wid * BPW

    for hdl in (
        pltpu.async_copy(hi_hbm.at[pl.ds(base, BPW)], hi_v, sem0),
        pltpu.async_copy(ri_hbm.at[pl.ds(base, BPW)], ri_v, sem0),
        pltpu.async_copy(ti_hbm.at[pl.ds(base, BPW)], ti_v, sem0),
    ):
        hdl.wait()

    bufs = [(hb0, rb0, tb0, sem0), (hb1, rb1, tb1, sem1)]

    def start(c):
        hb, rb, tb, sem = bufs[c % 2]
        sl = pl.ds(c * CH, CH)
        return (
            pltpu.async_copy(ent_hbm.at[hi_v.at[sl]], hb, sem),
            pltpu.async_copy(rel_hbm.at[ri_v.at[sl]], rb, sem),
            pltpu.async_copy(ent_hbm.at[ti_v.at[sl]], tb, sem),
        )

    pending = start(0)
    for c in range(NCH):
        for hdl in pending:
            hdl.wait()
        if c + 1 < NCH:
            pending = start(c + 1)
        hb, rb, tb, _ = bufs[c % 2]
        for g in range(CH // L):
            lanes = lax.iota(jnp.int32, L)
            rows = lanes + (g * L)
            zero = jnp.zeros((L,), jnp.float32)

            @plsc.parallel_loop(0, DIM // UNR, unroll=2, carry=(zero,) * UNR)
            def accs(i, acc_in, hb=hb, rb=rb, tb=tb, rows=rows, lanes=lanes):
                out = []
                for k in range(UNR):
                    cols = (lanes + (i * UNR + k)) & (DIM - 1)
                    h = plsc.load_gather(hb, [rows, cols])
                    r = plsc.load_gather(rb, [rows, cols])
                    t = plsc.load_gather(tb, [rows, cols])
                    out.append(acc_in[k] + jnp.abs(h + r - t))
                return tuple(out)

            acc = (accs[0] + accs[1]) + (accs[2] + accs[3])
            ob[pl.ds(c * CH + g * L, L)] = GAMMA - acc

    pltpu.sync_copy(ob, out_hbm.at[pl.ds(base, BPW)])


def kernel(sample, entity_embedding, relation_embedding):
    hi = sample[:, 0]
    ri = sample[:, 1]
    ti = sample[:, 2]
    score = _sc_score(hi, ri, ti, entity_embedding, relation_embedding)
    return score[:, None]


# speedup vs baseline: 1.1804x; 1.0657x over previous
import functools

import jax
import jax.numpy as jnp
from jax import lax
from jax.experimental import pallas as pl
from jax.experimental.pallas import tpu as pltpu
from jax.experimental.pallas import tpu_sc as plsc

B = 16384
DIM = 128
GAMMA = 12.0

NC = 2
NS = 16
L = 16
NW = NC * NS
BPW = B // NW
CH = 64
NCH = BPW // CH
NBUF = 3
UNR = 4

_mesh = plsc.VectorSubcoreMesh(core_axis_name="c", subcore_axis_name="s")


@functools.partial(
    pl.kernel,
    mesh=_mesh,
    out_type=jax.ShapeDtypeStruct((B,), jnp.float32),
    scratch_types=[
        pltpu.VMEM((BPW,), jnp.int32),
        pltpu.VMEM((BPW,), jnp.int32),
        pltpu.VMEM((BPW,), jnp.int32),
    ] + [
        pltpu.VMEM((CH, DIM), jnp.float32)
        for _ in range(3 * NBUF)
    ] + [
        pltpu.VMEM((BPW,), jnp.float32),
    ] + [pltpu.SemaphoreType.DMA for _ in range(NBUF)],
    compiler_params=pltpu.CompilerParams(needs_layout_passes=False),
)
def _sc_score(hi_hbm, ri_hbm, ti_hbm, ent_hbm, rel_hbm, out_hbm,
              hi_v, ri_v, ti_v,
              hb0, rb0, tb0, hb1, rb1, tb1, hb2, rb2, tb2, ob,
              sem0, sem1, sem2):
    cid = lax.axis_index("c")
    sid = lax.axis_index("s")
    wid = sid * NC + cid
    base = wid * BPW

    for hdl in (
        pltpu.async_copy(hi_hbm.at[pl.ds(base, BPW)], hi_v, sem0),
        pltpu.async_copy(ri_hbm.at[pl.ds(base, BPW)], ri_v, sem0),
        pltpu.async_copy(ti_hbm.at[pl.ds(base, BPW)], ti_v, sem0),
    ):
        hdl.wait()

    bufs = [(hb0, rb0, tb0, sem0), (hb1, rb1, tb1, sem1),
            (hb2, rb2, tb2, sem2)]

    def start(c):
        hb, rb, tb, sem = bufs[c % NBUF]
        sl = pl.ds(c * CH, CH)
        return (
            pltpu.async_copy(ent_hbm.at[hi_v.at[sl]], hb, sem),
            pltpu.async_copy(rel_hbm.at[ri_v.at[sl]], rb, sem),
            pltpu.async_copy(ent_hbm.at[ti_v.at[sl]], tb, sem),
        )

    pending = [start(0), start(1)]
    for c in range(NCH):
        for hdl in pending.pop(0):
            hdl.wait()
        if c + 2 < NCH:
            pending.append(start(c + 2))
        hb, rb, tb, _ = bufs[c % NBUF]
        for g in range(CH // L):
            lanes = lax.iota(jnp.int32, L)
            rows = lanes + (g * L)
            zero = jnp.zeros((L,), jnp.float32)

            @plsc.parallel_loop(0, DIM // UNR, unroll=2, carry=(zero,) * UNR)
            def accs(i, acc_in, hb=hb, rb=rb, tb=tb, rows=rows, lanes=lanes):
                out = []
                for k in range(UNR):
                    cols = (lanes + (i * UNR + k)) & (DIM - 1)
                    h = plsc.load_gather(hb, [rows, cols])
                    r = plsc.load_gather(rb, [rows, cols])
                    t = plsc.load_gather(tb, [rows, cols])
                    out.append(acc_in[k] + jnp.abs(h + r - t))
                return tuple(out)

            acc = (accs[0] + accs[1]) + (accs[2] + accs[3])
            ob[pl.ds(c * CH + g * L, L)] = GAMMA - acc

    pltpu.sync_copy(ob, out_hbm.at[pl.ds(base, BPW)])


def kernel(sample, entity_embedding, relation_embedding):
    hi = sample[:, 0]
    ri = sample[:, 1]
    ti = sample[:, 2]
    score = _sc_score(hi, ri, ti, entity_embedding, relation_embedding)
    return score[:, None]


# speedup vs baseline: 1.1903x; 1.0084x over previous
import functools

import jax
import jax.numpy as jnp
from jax import lax
from jax.experimental import pallas as pl
from jax.experimental.pallas import tpu as pltpu
from jax.experimental.pallas import tpu_sc as plsc

B = 16384
DIM = 128
GAMMA = 12.0

NC = 2
NS = 16
L = 16
NW = NC * NS
BPW = B // NW
CH = 32
NCH = BPW // CH
NBUF = 4
UNR = 4

_mesh = plsc.VectorSubcoreMesh(core_axis_name="c", subcore_axis_name="s")


@functools.partial(
    pl.kernel,
    mesh=_mesh,
    out_type=jax.ShapeDtypeStruct((B,), jnp.float32),
    scratch_types=[
        pltpu.VMEM((BPW,), jnp.int32),
        pltpu.VMEM((BPW,), jnp.int32),
        pltpu.VMEM((BPW,), jnp.int32),
    ] + [
        pltpu.VMEM((CH, DIM), jnp.float32)
        for _ in range(3 * NBUF)
    ] + [
        pltpu.VMEM((BPW,), jnp.float32),
    ] + [pltpu.SemaphoreType.DMA for _ in range(NBUF)],
    compiler_params=pltpu.CompilerParams(needs_layout_passes=False),
)
def _sc_score(hi_hbm, ri_hbm, ti_hbm, ent_hbm, rel_hbm, out_hbm,
              hi_v, ri_v, ti_v,
              hb0, rb0, tb0, hb1, rb1, tb1, hb2, rb2, tb2, hb3, rb3, tb3,
              ob, sem0, sem1, sem2, sem3):
    cid = lax.axis_index("c")
    sid = lax.axis_index("s")
    wid = sid * NC + cid
    base = wid * BPW

    for hdl in (
        pltpu.async_copy(hi_hbm.at[pl.ds(base, BPW)], hi_v, sem0),
        pltpu.async_copy(ri_hbm.at[pl.ds(base, BPW)], ri_v, sem0),
        pltpu.async_copy(ti_hbm.at[pl.ds(base, BPW)], ti_v, sem0),
    ):
        hdl.wait()

    bufs = [(hb0, rb0, tb0, sem0), (hb1, rb1, tb1, sem1),
            (hb2, rb2, tb2, sem2), (hb3, rb3, tb3, sem3)]

    def start(c):
        hb, rb, tb, sem = bufs[c % NBUF]
        sl = pl.ds(c * CH, CH)
        return (
            pltpu.async_copy(ent_hbm.at[hi_v.at[sl]], hb, sem),
            pltpu.async_copy(rel_hbm.at[ri_v.at[sl]], rb, sem),
            pltpu.async_copy(ent_hbm.at[ti_v.at[sl]], tb, sem),
        )

    pending = [start(0), start(1), start(2)]
    for c in range(NCH):
        for hdl in pending.pop(0):
            hdl.wait()
        if c + 3 < NCH:
            pending.append(start(c + 3))
        hb, rb, tb, _ = bufs[c % NBUF]
        for g in range(CH // L):
            lanes = lax.iota(jnp.int32, L)
            rows = lanes + (g * L)
            zero = jnp.zeros((L,), jnp.float32)

            @plsc.parallel_loop(0, DIM // UNR, unroll=2, carry=(zero,) * UNR)
            def accs(i, acc_in, hb=hb, rb=rb, tb=tb, rows=rows, lanes=lanes):
                out = []
                for k in range(UNR):
                    cols = (lanes + (i * UNR + k)) & (DIM - 1)
                    h = plsc.load_gather(hb, [rows, cols])
                    r = plsc.load_gather(rb, [rows, cols])
                    t = plsc.load_gather(tb, [rows, cols])
                    out.append(acc_in[k] + jnp.abs(h + r - t))
                return tuple(out)

            acc = (accs[0] + accs[1]) + (accs[2] + accs[3])
            ob[pl.ds(c * CH + g * L, L)] = GAMMA - acc

    pltpu.sync_copy(ob, out_hbm.at[pl.ds(base, BPW)])


def kernel(sample, entity_embedding, relation_embedding):
    hi = sample[:, 0]
    ri = sample[:, 1]
    ti = sample[:, 2]
    score = _sc_score(hi, ri, ti, entity_embedding, relation_embedding)
    return score[:, None]


# speedup vs baseline: 1.1970x; 1.0056x over previous
import functools

import jax
import jax.numpy as jnp
from jax import lax
from jax.experimental import pallas as pl
from jax.experimental.pallas import tpu as pltpu
from jax.experimental.pallas import tpu_sc as plsc

B = 16384
DIM = 128
GAMMA = 12.0

NC = 2
NS = 16
L = 16
NW = NC * NS
BPW = B // NW
CH = 32
NCH = BPW // CH
NBUF = 4
UNR = 4
NGC = B // CH

_mesh = plsc.VectorSubcoreMesh(core_axis_name="c", subcore_axis_name="s")


@functools.partial(
    pl.kernel,
    mesh=_mesh,
    out_type=jax.ShapeDtypeStruct((B,), jnp.float32),
    scratch_types=[
        pltpu.VMEM((NCH, 2 * CH), jnp.int32),
        pltpu.VMEM((BPW,), jnp.int32),
    ] + [
        pltpu.VMEM((2 * CH, DIM), jnp.float32)
        for _ in range(NBUF)
    ] + [
        pltpu.VMEM((CH, DIM), jnp.float32)
        for _ in range(NBUF)
    ] + [
        pltpu.VMEM((BPW,), jnp.float32),
    ] + [pltpu.SemaphoreType.DMA for _ in range(NBUF)],
    compiler_params=pltpu.CompilerParams(needs_layout_passes=False),
)
def _sc_score(ht_hbm, ri_hbm, ent_hbm, rel_hbm, out_hbm,
              ht_v, ri_v, htb0, htb1, htb2, htb3, rb0, rb1, rb2, rb3,
              ob, sem0, sem1, sem2, sem3):
    cid = lax.axis_index("c")
    sid = lax.axis_index("s")
    wid = sid * NC + cid
    base = wid * BPW

    for hdl in (
        pltpu.async_copy(ht_hbm.at[pl.ds(wid * NCH, NCH)], ht_v, sem0),
        pltpu.async_copy(ri_hbm.at[pl.ds(base, BPW)], ri_v, sem0),
    ):
        hdl.wait()

    bufs = [(htb0, rb0, sem0), (htb1, rb1, sem1),
            (htb2, rb2, sem2), (htb3, rb3, sem3)]

    def start(c):
        htb, rb, sem = bufs[c % NBUF]
        return (
            pltpu.async_copy(ent_hbm.at[ht_v.at[c]], htb, sem),
            pltpu.async_copy(rel_hbm.at[ri_v.at[pl.ds(c * CH, CH)]], rb, sem),
        )

    pending = [start(0), start(1), start(2)]
    for c in range(NCH):
        for hdl in pending.pop(0):
            hdl.wait()
        if c + 3 < NCH:
            pending.append(start(c + 3))
        htb, rb, _ = bufs[c % NBUF]
        for g in range(CH // L):
            lanes = lax.iota(jnp.int32, L)
            rows = lanes + (g * L)
            trows = rows + CH
            zero = jnp.zeros((L,), jnp.float32)

            @plsc.parallel_loop(0, DIM // UNR, unroll=2, carry=(zero,) * UNR)
            def accs(i, acc_in, htb=htb, rb=rb, rows=rows, trows=trows,
                     lanes=lanes):
                out = []
                for k in range(UNR):
                    cols = (lanes + (i * UNR + k)) & (DIM - 1)
                    h = plsc.load_gather(htb, [rows, cols])
                    r = plsc.load_gather(rb, [rows, cols])
                    t = plsc.load_gather(htb, [trows, cols])
                    out.append(acc_in[k] + jnp.abs(h + r - t))
                return tuple(out)

            acc = (accs[0] + accs[1]) + (accs[2] + accs[3])
            ob[pl.ds(c * CH + g * L, L)] = GAMMA - acc

    pltpu.sync_copy(ob, out_hbm.at[pl.ds(base, BPW)])


def kernel(sample, entity_embedding, relation_embedding):
    ht = jnp.concatenate(
        [sample[:, 0].reshape(NGC, CH), sample[:, 2].reshape(NGC, CH)],
        axis=1)
    ri = sample[:, 1]
    score = _sc_score(ht, ri, entity_embedding, relation_embedding)
    return score[:, None]
